# baseline (speedup 1.0000x reference)
import functools

import jax
import jax.numpy as jnp
from jax import lax
from jax.experimental import pallas as pl
from jax.experimental.pallas import tpu as pltpu
from jax.experimental.pallas import tpu_sc as plsc

_NC = 2
_NS = 16
_NW = _NC * _NS

_CHUNK = 1280
_BLK = 10240


def _make_gather(tok, hid):
    assert tok % (_NW * _CHUNK) == 0
    tok_per_w = tok // _NW
    n_chunk = tok_per_w // _CHUNK
    mesh = plsc.VectorSubcoreMesh(core_axis_name="c", subcore_axis_name="s")

    @functools.partial(
        pl.kernel,
        out_type=jax.ShapeDtypeStruct((tok, hid), jnp.bfloat16),
        mesh=mesh,
        scratch_types=[
            pltpu.VMEM((tok_per_w,), jnp.int32),
            pltpu.VMEM((_CHUNK, hid), jnp.bfloat16),
            pltpu.VMEM((_CHUNK, hid), jnp.bfloat16),
            pltpu.SemaphoreType.DMA,
            pltpu.SemaphoreType.DMA,
            pltpu.SemaphoreType.DMA,
            pltpu.SemaphoreType.DMA,
        ],
        compiler_params=pltpu.CompilerParams(use_tc_tiling_on_sc=False),
    )
    def gather(idx_hbm, table_hbm, out_hbm, idx_v, rows_v0, rows_v1,
               sg0, sg1, sw0, sw1):
        wid = lax.axis_index("s") * _NC + lax.axis_index("c")
        base = wid * tok_per_w
        pltpu.sync_copy(idx_hbm.at[pl.ds(base, tok_per_w)], idx_v)

        rows = (rows_v0, rows_v1)
        sg = (sg0, sg1)
        sw = (sw0, sw1)
        writes = [None, None]
        for i in range(n_chunk):
            b = i % 2
            if writes[b] is not None:
                writes[b].wait()
            g = pltpu.async_copy(
                table_hbm.at[idx_v.at[pl.ds(i * _CHUNK, _CHUNK)]],
                rows[b], sg[b],
            )
            g.wait()
            writes[b] = pltpu.async_copy(
                rows[b], out_hbm.at[pl.ds(base + i * _CHUNK, _CHUNK)], sw[b]
            )
        for w in writes:
            if w is not None:
                w.wait()

    return gather


def _matmul_body(emb_ref, w4_ref, out_ref):
    res = jnp.dot(emb_ref[...], w4_ref[...], preferred_element_type=jnp.float32)
    for i in range(4):
        out_ref[pl.ds(i * _BLK, _BLK), :] = res[:, i * 128:(i + 1) * 128]


def _project(emb_wide, w4, tok, emb_dim):
    rows = emb_wide.shape[0]
    grid = rows // _BLK
    return pl.pallas_call(
        _matmul_body,
        grid=(grid,),
        in_specs=[
            pl.BlockSpec((_BLK, 128), lambda i: (i, 0)),
            pl.BlockSpec((128, 4 * 128), lambda i: (0, 0)),
        ],
        out_specs=pl.BlockSpec((4 * _BLK, emb_dim), lambda i: (i, 0)),
        out_shape=jax.ShapeDtypeStruct((tok, emb_dim), jnp.float32),
    )(emb_wide, w4)


def kernel(x, em_weight, fc_weight):
    b, seq = x.shape
    n_emb, hid = em_weight.shape
    emb_dim = fc_weight.shape[0]
    tok = b * seq
    grp = 4 * _BLK

    idx = x.reshape(tok // grp, 4, _BLK).transpose(0, 2, 1).reshape(tok)

    em16 = em_weight.astype(jnp.bfloat16)
    emb = _make_gather(tok, hid)(idx, em16)
    emb_wide = emb.reshape(tok // 4, 4 * hid)

    fct = fc_weight.T.astype(jnp.bfloat16)
    eye4 = jnp.eye(4, dtype=fct.dtype)
    w4 = jnp.einsum("gh,ke->gkhe", eye4, fct).reshape(4 * hid, 4 * emb_dim)

    out = _project(emb_wide, w4, tok, emb_dim)
    return out.reshape(b, seq, emb_dim)


# speedup vs baseline: 1.4721x; 1.4721x over previous
import functools

import jax
import jax.numpy as jnp
from jax import lax
from jax.experimental import pallas as pl
from jax.experimental.pallas import tpu as pltpu
from jax.experimental.pallas import tpu_sc as plsc

_NC = 2
_NS = 16
_NW = _NC * _NS

_CHUNK = 1280
_BLK = 10240


def _make_gather(tok, hid):
    assert tok % (_NW * _CHUNK) == 0
    tok_per_w = tok // _NW
    n_chunk = tok_per_w // _CHUNK
    mesh = plsc.VectorSubcoreMesh(core_axis_name="c", subcore_axis_name="s")

    @functools.partial(
        pl.kernel,
        out_type=jax.ShapeDtypeStruct((tok, hid), jnp.float32),
        mesh=mesh,
        scratch_types=[
            pltpu.VMEM((tok_per_w,), jnp.int32),
            pltpu.VMEM((_CHUNK, hid), jnp.float32),
            pltpu.VMEM((_CHUNK, hid), jnp.float32),
            pltpu.SemaphoreType.DMA,
            pltpu.SemaphoreType.DMA,
            pltpu.SemaphoreType.DMA,
            pltpu.SemaphoreType.DMA,
        ],
        compiler_params=pltpu.CompilerParams(use_tc_tiling_on_sc=False),
    )
    def gather(idx_hbm, table_hbm, out_hbm, idx_v, rows_v0, rows_v1,
               sg0, sg1, sw0, sw1):
        wid = lax.axis_index("s") * _NC + lax.axis_index("c")
        base = wid * tok_per_w
        pltpu.sync_copy(idx_hbm.at[pl.ds(base, tok_per_w)], idx_v)

        rows = (rows_v0, rows_v1)
        sg = (sg0, sg1)
        sw = (sw0, sw1)
        writes = [None, None]
        for i in range(n_chunk):
            b = i % 2
            if writes[b] is not None:
                writes[b].wait()
            g = pltpu.async_copy(
                table_hbm.at[idx_v.at[pl.ds(i * _CHUNK, _CHUNK)]],
                rows[b], sg[b],
            )
            g.wait()
            writes[b] = pltpu.async_copy(
                rows[b], out_hbm.at[pl.ds(base + i * _CHUNK, _CHUNK)], sw[b]
            )
        for w in writes:
            if w is not None:
                w.wait()

    return gather


def _matmul_body(emb_ref, w4_ref, out_ref):
    res = jnp.dot(emb_ref[...], w4_ref[...], preferred_element_type=jnp.float32)
    for i in range(4):
        out_ref[pl.ds(i * _BLK, _BLK), :] = res[:, i * 128:(i + 1) * 128]


def _project(emb_wide, w4, tok, emb_dim):
    rows = emb_wide.shape[0]
    grid = rows // _BLK
    return pl.pallas_call(
        _matmul_body,
        grid=(grid,),
        in_specs=[
            pl.BlockSpec((_BLK, 128), lambda i: (i, 0)),
            pl.BlockSpec((128, 4 * 128), lambda i: (0, 0)),
        ],
        out_specs=pl.BlockSpec((4 * _BLK, emb_dim), lambda i: (i, 0)),
        out_shape=jax.ShapeDtypeStruct((tok, emb_dim), jnp.float32),
    )(emb_wide, w4)


def kernel(x, em_weight, fc_weight):
    b, seq = x.shape
    n_emb, hid = em_weight.shape
    emb_dim = fc_weight.shape[0]
    tok = b * seq
    grp = 4 * _BLK

    idx = x.reshape(tok // grp, 4, _BLK).transpose(0, 2, 1).reshape(tok)

    emb = _make_gather(tok, hid)(idx, em_weight)
    emb_wide = emb.reshape(tok // 4, 4 * hid)

    fct = fc_weight.T
    eye4 = jnp.eye(4, dtype=fct.dtype)
    w4 = jnp.einsum("gh,ke->gkhe", eye4, fct).reshape(4 * hid, 4 * emb_dim)

    out = _project(emb_wide, w4, tok, emb_dim)
    return out.reshape(b, seq, emb_dim)
